# baseline (speedup 1.0000x reference)
import functools

import jax
import jax.numpy as jnp
from jax import lax
from jax.experimental import pallas as pl
from jax.experimental.pallas import tpu as pltpu
from jax.experimental.pallas import tpu_sc as plsc

EMB = 64
HOM = 3
_IDX_W = 128
_CHUNK = 512
_PB = 512


def _pack_body(x1_ref, x2_ref, out_ref):
    x1 = x1_ref[...]
    x2 = x2_ref[...]
    eye = jnp.eye(EMB, dtype=jnp.float32)
    dn = (((0,), (0,)), ((), ()))
    x1t = lax.dot_general(x1, eye, dn, preferred_element_type=jnp.float32)
    x2t = lax.dot_general(x2, eye, dn, preferred_element_type=jnp.float32)
    out_ref[...] = jnp.concatenate([x1t, x2t], axis=1)


@functools.lru_cache(maxsize=None)
def _make_tc_pack(V):
    K0 = (V // 2) // _PB * _PB
    R = V - K0
    nb = (R + _PB - 1) // _PB
    g0 = K0 // _PB
    return pl.pallas_call(
        _pack_body,
        grid=(nb,),
        in_specs=[
            pl.BlockSpec((EMB, _PB), lambda g: (0, g)),
            pl.BlockSpec((EMB, _PB), lambda g, _g0=g0: (0, _g0 + g)),
        ],
        out_specs=pl.BlockSpec((_PB, 2 * EMB), lambda g: (g, 0)),
        out_shape=jax.ShapeDtypeStruct((R, 2 * EMB), jnp.float32),
    )


@functools.lru_cache(maxsize=None)
def _make_sc_gather(V2, D2, B):
    info = plsc.get_sparse_core_info()
    NW = info.num_cores * info.num_subcores
    NC = info.num_cores
    b_per_w = B // NW
    n_chunks = b_per_w // _CHUNK
    streams_per_chunk = _CHUNK // _IDX_W
    idx_rows_per_w = b_per_w // _IDX_W
    assert b_per_w * NW == B and n_chunks * _CHUNK == b_per_w
    mesh = plsc.VectorSubcoreMesh(core_axis_name="c", subcore_axis_name="s")

    @functools.partial(
        pl.kernel,
        mesh=mesh,
        out_type=jax.ShapeDtypeStruct((B, D2), jnp.float32),
        scratch_types=[
            pltpu.VMEM((idx_rows_per_w, _IDX_W), jnp.int32),
            pltpu.VMEM((_CHUNK, D2), jnp.float32),
            pltpu.SemaphoreType.DMA,
        ],
        compiler_params=pltpu.CompilerParams(use_tc_tiling_on_sc=True),
    )
    def gather_k(table_hbm, idx_hbm, out_hbm, idx_v, rows_v, sem):
        wid = lax.axis_index("s") * NC + lax.axis_index("c")
        base = wid * b_per_w
        pltpu.sync_copy(idx_hbm.at[pl.ds(wid * idx_rows_per_w, idx_rows_per_w)], idx_v)
        for ch in range(n_chunks):
            copies = []
            for j in range(streams_per_chunk):
                copies.append(
                    pltpu.async_copy(
                        table_hbm.at[idx_v.at[ch * streams_per_chunk + j]],
                        rows_v.at[pl.ds(j * _IDX_W, _IDX_W)],
                        sem,
                    )
                )
            for cp in copies:
                cp.wait()
            pltpu.sync_copy(rows_v, out_hbm.at[pl.ds(base + ch * _CHUNK, _CHUNK)])

    return gather_k


def _dense_body(c_ref, d_ref, pc_ref, pd_ref, wt_ref, b_ref, out_ref):
    c2 = c_ref[...]
    d2 = d_ref[...]
    pc = pc_ref[...] != 0
    pd = pd_ref[...] != 0
    c = jnp.where(pc, c2[:, EMB:], c2[:, :EMB])
    d = jnp.where(pd, d2[:, EMB:], d2[:, :EMB])
    tgt = jax.nn.sigmoid(d)
    acc = None
    for k in range(HOM):
        est = jax.nn.sigmoid(
            jnp.dot(c, wt_ref[k], preferred_element_type=jnp.float32) + b_ref[k]
        )
        diff = est - tgt
        dist = jnp.sqrt(jnp.sum(diff * diff, axis=1, keepdims=True) + 1e-12)
        acc = dist if acc is None else acc + dist
    out_ref[...] = acc * (1.0 / HOM)


@functools.lru_cache(maxsize=None)
def _make_tc_dense(B1, BB):
    nb = B1 // BB
    assert nb * BB == B1
    return pl.pallas_call(
        _dense_body,
        grid=(nb,),
        in_specs=[
            pl.BlockSpec((BB, 2 * EMB), lambda g: (g, 0)),
            pl.BlockSpec((BB, 2 * EMB), lambda g: (g + nb, 0)),
            pl.BlockSpec((BB, 1), lambda g: (g, 0)),
            pl.BlockSpec((BB, 1), lambda g: (g, 0)),
            pl.BlockSpec((HOM, EMB, EMB), lambda g: (0, 0, 0)),
            pl.BlockSpec((HOM, 1, EMB), lambda g: (0, 0, 0)),
        ],
        out_specs=pl.BlockSpec((BB, 1), lambda g: (g, 0)),
        out_shape=jax.ShapeDtypeStruct((B1, 1), jnp.float32),
    )


def kernel(data, idx, embed, embed_rel, hom_W, hom_b):
    B1 = data.shape[0]
    V, D = embed.shape
    K0 = (V // 2) // _PB * _PB
    table2 = _make_tc_pack(V)(embed.T, embed.T)
    idx_all = jnp.concatenate([data[:, 0], data[:, 1]])
    idx2 = jnp.where(idx_all < K0, idx_all, idx_all - K0).reshape(-1, _IDX_W)
    cd = _make_sc_gather(V - K0, 2 * D, 2 * B1)(table2, idx2)
    pc = (data[:, 0] >= K0).astype(jnp.int32).reshape(B1, 1)
    pd = (data[:, 1] >= K0).astype(jnp.int32).reshape(B1, 1)
    wt = jnp.transpose(hom_W, (0, 2, 1))
    b3 = hom_b[:, None, :]
    loss = _make_tc_dense(B1, 512)(cd, cd, pc, pd, wt, b3)[:, 0]
    guard = jnp.where(jnp.asarray(idx) != 0, jnp.float32(jnp.nan), jnp.float32(0.0))
    return loss + guard


# speedup vs baseline: 2.3482x; 2.3482x over previous
import functools

import jax
import jax.numpy as jnp
from jax import lax
from jax.experimental import pallas as pl
from jax.experimental.pallas import tpu as pltpu
from jax.experimental.pallas import tpu_sc as plsc

EMB = 64
HOM = 3
_IDX_W = 128
_CHUNK = 512
_PB = 4096


def _pack_body(x1_ref, x2_ref, out_ref):
    x1 = x1_ref[...]
    x2 = x2_ref[...]
    out_ref[...] = jnp.concatenate([x1.T, x2.T], axis=1)


@functools.lru_cache(maxsize=None)
def _make_tc_pack(V):
    K0 = (V // 2) // _PB * _PB
    R = V - K0
    nb = (R + _PB - 1) // _PB
    g0 = K0 // _PB
    return pl.pallas_call(
        _pack_body,
        grid=(nb,),
        in_specs=[
            pl.BlockSpec((EMB, _PB), lambda g: (0, g)),
            pl.BlockSpec((EMB, _PB), lambda g, _g0=g0: (0, _g0 + g)),
        ],
        out_specs=pl.BlockSpec((_PB, 2 * EMB), lambda g: (g, 0)),
        out_shape=jax.ShapeDtypeStruct((R, 2 * EMB), jnp.float32),
    )


@functools.lru_cache(maxsize=None)
def _make_sc_gather(V2, D2, B):
    info = plsc.get_sparse_core_info()
    NW = info.num_cores * info.num_subcores
    NC = info.num_cores
    b_per_w = B // NW
    n_chunks = b_per_w // _CHUNK
    streams_per_chunk = _CHUNK // _IDX_W
    idx_rows_per_w = b_per_w // _IDX_W
    assert b_per_w * NW == B and n_chunks * _CHUNK == b_per_w
    mesh = plsc.VectorSubcoreMesh(core_axis_name="c", subcore_axis_name="s")

    @functools.partial(
        pl.kernel,
        mesh=mesh,
        out_type=jax.ShapeDtypeStruct((B, D2), jnp.float32),
        scratch_types=[
            pltpu.VMEM((idx_rows_per_w, _IDX_W), jnp.int32),
            pltpu.VMEM((_CHUNK, D2), jnp.float32),
            pltpu.SemaphoreType.DMA,
        ],
        compiler_params=pltpu.CompilerParams(use_tc_tiling_on_sc=True),
    )
    def gather_k(table_hbm, idx_hbm, out_hbm, idx_v, rows_v, sem):
        wid = lax.axis_index("s") * NC + lax.axis_index("c")
        base = wid * b_per_w
        pltpu.sync_copy(idx_hbm.at[pl.ds(wid * idx_rows_per_w, idx_rows_per_w)], idx_v)
        for ch in range(n_chunks):
            copies = []
            for j in range(streams_per_chunk):
                copies.append(
                    pltpu.async_copy(
                        table_hbm.at[idx_v.at[ch * streams_per_chunk + j]],
                        rows_v.at[pl.ds(j * _IDX_W, _IDX_W)],
                        sem,
                    )
                )
            for cp in copies:
                cp.wait()
            pltpu.sync_copy(rows_v, out_hbm.at[pl.ds(base + ch * _CHUNK, _CHUNK)])

    return gather_k


def _dense_body(c_ref, d_ref, pc_ref, pd_ref, wt_ref, b_ref, out_ref):
    c2 = c_ref[...]
    d2 = d_ref[...]
    pc = pc_ref[...] != 0
    pd = pd_ref[...] != 0
    c = jnp.where(pc, c2[:, EMB:], c2[:, :EMB])
    d = jnp.where(pd, d2[:, EMB:], d2[:, :EMB])
    tgt = jax.nn.sigmoid(d)
    acc = None
    for k in range(HOM):
        est = jax.nn.sigmoid(
            jnp.dot(c, wt_ref[k], preferred_element_type=jnp.float32) + b_ref[k]
        )
        diff = est - tgt
        dist = jnp.sqrt(jnp.sum(diff * diff, axis=1, keepdims=True) + 1e-12)
        acc = dist if acc is None else acc + dist
    out_ref[...] = acc * (1.0 / HOM)


@functools.lru_cache(maxsize=None)
def _make_tc_dense(B1, BB):
    nb = B1 // BB
    assert nb * BB == B1
    return pl.pallas_call(
        _dense_body,
        grid=(nb,),
        in_specs=[
            pl.BlockSpec((BB, 2 * EMB), lambda g: (g, 0)),
            pl.BlockSpec((BB, 2 * EMB), lambda g: (g + nb, 0)),
            pl.BlockSpec((BB, 1), lambda g: (g, 0)),
            pl.BlockSpec((BB, 1), lambda g: (g, 0)),
            pl.BlockSpec((HOM, EMB, EMB), lambda g: (0, 0, 0)),
            pl.BlockSpec((HOM, 1, EMB), lambda g: (0, 0, 0)),
        ],
        out_specs=pl.BlockSpec((BB, 1), lambda g: (g, 0)),
        out_shape=jax.ShapeDtypeStruct((B1, 1), jnp.float32),
    )


def kernel(data, idx, embed, embed_rel, hom_W, hom_b):
    B1 = data.shape[0]
    V, D = embed.shape
    K0 = (V // 2) // _PB * _PB
    table2 = _make_tc_pack(V)(embed.T, embed.T)
    idx_all = jnp.concatenate([data[:, 0], data[:, 1]])
    idx2 = jnp.where(idx_all < K0, idx_all, idx_all - K0).reshape(-1, _IDX_W)
    cd = _make_sc_gather(V - K0, 2 * D, 2 * B1)(table2, idx2)
    pc = (data[:, 0] >= K0).astype(jnp.int32).reshape(B1, 1)
    pd = (data[:, 1] >= K0).astype(jnp.int32).reshape(B1, 1)
    wt = jnp.transpose(hom_W, (0, 2, 1))
    b3 = hom_b[:, None, :]
    loss = _make_tc_dense(B1, 512)(cd, cd, pc, pd, wt, b3)[:, 0]
    guard = jnp.where(jnp.asarray(idx) != 0, jnp.float32(jnp.nan), jnp.float32(0.0))
    return loss + guard


# speedup vs baseline: 2.5906x; 1.1032x over previous
import functools

import jax
import jax.numpy as jnp
from jax import lax
from jax.experimental import pallas as pl
from jax.experimental.pallas import tpu as pltpu
from jax.experimental.pallas import tpu_sc as plsc

EMB = 64
HOM = 3
_IDX_W = 128
_CHUNK = 512
_PB = 8192


def _pack_body(x1_ref, x2_ref, out_ref):
    x1 = x1_ref[...]
    x2 = x2_ref[...]
    out_ref[...] = jnp.concatenate([x1.T, x2.T], axis=1)


@functools.lru_cache(maxsize=None)
def _make_tc_pack(V):
    K0 = (V // 2) // _PB * _PB
    R = V - K0
    nb = (R + _PB - 1) // _PB
    g0 = K0 // _PB
    return pl.pallas_call(
        _pack_body,
        grid=(nb,),
        in_specs=[
            pl.BlockSpec((EMB, _PB), lambda g: (0, g)),
            pl.BlockSpec((EMB, _PB), lambda g, _g0=g0: (0, _g0 + g)),
        ],
        out_specs=pl.BlockSpec((_PB, 2 * EMB), lambda g: (g, 0)),
        out_shape=jax.ShapeDtypeStruct((R, 2 * EMB), jnp.float32),
    )


@functools.lru_cache(maxsize=None)
def _make_sc_gather(V2, D2, B):
    info = plsc.get_sparse_core_info()
    NW = info.num_cores * info.num_subcores
    NC = info.num_cores
    b_per_w = B // NW
    n_chunks = b_per_w // _CHUNK
    streams_per_chunk = _CHUNK // _IDX_W
    idx_rows_per_w = b_per_w // _IDX_W
    assert b_per_w * NW == B and n_chunks * _CHUNK == b_per_w
    mesh = plsc.VectorSubcoreMesh(core_axis_name="c", subcore_axis_name="s")

    @functools.partial(
        pl.kernel,
        mesh=mesh,
        out_type=jax.ShapeDtypeStruct((B, D2), jnp.float32),
        scratch_types=[
            pltpu.VMEM((idx_rows_per_w, _IDX_W), jnp.int32),
            pltpu.VMEM((_CHUNK, D2), jnp.float32),
            pltpu.SemaphoreType.DMA,
        ],
        compiler_params=pltpu.CompilerParams(use_tc_tiling_on_sc=True),
    )
    def gather_k(table_hbm, idx_hbm, out_hbm, idx_v, rows_v, sem):
        wid = lax.axis_index("s") * NC + lax.axis_index("c")
        base = wid * b_per_w
        pltpu.sync_copy(idx_hbm.at[pl.ds(wid * idx_rows_per_w, idx_rows_per_w)], idx_v)
        for ch in range(n_chunks):
            copies = []
            for j in range(streams_per_chunk):
                copies.append(
                    pltpu.async_copy(
                        table_hbm.at[idx_v.at[ch * streams_per_chunk + j]],
                        rows_v.at[pl.ds(j * _IDX_W, _IDX_W)],
                        sem,
                    )
                )
            for cp in copies:
                cp.wait()
            pltpu.sync_copy(rows_v, out_hbm.at[pl.ds(base + ch * _CHUNK, _CHUNK)])

    return gather_k


def _dense_body(c_ref, d_ref, pc_ref, pd_ref, wt_ref, b_ref, out_ref):
    c2 = c_ref[...]
    d2 = d_ref[...]
    pc = pc_ref[...] != 0
    pd = pd_ref[...] != 0
    c = jnp.where(pc, c2[:, EMB:], c2[:, :EMB])
    d = jnp.where(pd, d2[:, EMB:], d2[:, :EMB])
    tgt = jax.nn.sigmoid(d)
    acc = None
    for k in range(HOM):
        est = jax.nn.sigmoid(
            jnp.dot(c, wt_ref[k], preferred_element_type=jnp.float32) + b_ref[k]
        )
        diff = est - tgt
        dist = jnp.sqrt(jnp.sum(diff * diff, axis=1, keepdims=True) + 1e-12)
        acc = dist if acc is None else acc + dist
    out_ref[...] = acc * (1.0 / HOM)


@functools.lru_cache(maxsize=None)
def _make_tc_dense(B1, BB):
    nb = B1 // BB
    assert nb * BB == B1
    return pl.pallas_call(
        _dense_body,
        grid=(nb,),
        in_specs=[
            pl.BlockSpec((BB, 2 * EMB), lambda g: (g, 0)),
            pl.BlockSpec((BB, 2 * EMB), lambda g: (g + nb, 0)),
            pl.BlockSpec((BB, 1), lambda g: (g, 0)),
            pl.BlockSpec((BB, 1), lambda g: (g, 0)),
            pl.BlockSpec((HOM, EMB, EMB), lambda g: (0, 0, 0)),
            pl.BlockSpec((HOM, 1, EMB), lambda g: (0, 0, 0)),
        ],
        out_specs=pl.BlockSpec((BB, 1), lambda g: (g, 0)),
        out_shape=jax.ShapeDtypeStruct((B1, 1), jnp.float32),
    )


def kernel(data, idx, embed, embed_rel, hom_W, hom_b):
    B1 = data.shape[0]
    V, D = embed.shape
    K0 = (V // 2) // _PB * _PB
    table2 = _make_tc_pack(V)(embed.T, embed.T)
    idx_all = jnp.concatenate([data[:, 0], data[:, 1]])
    idx2 = jnp.where(idx_all < K0, idx_all, idx_all - K0).reshape(-1, _IDX_W)
    cd = _make_sc_gather(V - K0, 2 * D, 2 * B1)(table2, idx2)
    pc = (data[:, 0] >= K0).astype(jnp.int32).reshape(B1, 1)
    pd = (data[:, 1] >= K0).astype(jnp.int32).reshape(B1, 1)
    wt = jnp.transpose(hom_W, (0, 2, 1))
    b3 = hom_b[:, None, :]
    loss = _make_tc_dense(B1, 512)(cd, cd, pc, pd, wt, b3)[:, 0]
    guard = jnp.where(jnp.asarray(idx) != 0, jnp.float32(jnp.nan), jnp.float32(0.0))
    return loss + guard


# speedup vs baseline: 2.7503x; 1.0616x over previous
import functools

import jax
import jax.numpy as jnp
from jax import lax
from jax.experimental import pallas as pl
from jax.experimental.pallas import tpu as pltpu
from jax.experimental.pallas import tpu_sc as plsc

EMB = 64
HOM = 3
_IDX_W = 128
_CHUNK = 256
_PB = 16384


def _pack_body(x1_ref, x2_ref, out_ref):
    x1 = x1_ref[...]
    x2 = x2_ref[...]
    out_ref[...] = jnp.concatenate([x1.T, x2.T], axis=1)


@functools.lru_cache(maxsize=None)
def _make_tc_pack(V):
    K0 = (V // 2) // _PB * _PB
    R = V - K0
    nb = (R + _PB - 1) // _PB
    g0 = K0 // _PB
    return pl.pallas_call(
        _pack_body,
        grid=(nb,),
        in_specs=[
            pl.BlockSpec((EMB, _PB), lambda g: (0, g)),
            pl.BlockSpec((EMB, _PB), lambda g, _g0=g0: (0, _g0 + g)),
        ],
        out_specs=pl.BlockSpec((_PB, 2 * EMB), lambda g: (g, 0)),
        out_shape=jax.ShapeDtypeStruct((R, 2 * EMB), jnp.float32),
    )


@functools.lru_cache(maxsize=None)
def _make_sc_gather(V2, D2, B):
    info = plsc.get_sparse_core_info()
    NW = info.num_cores * info.num_subcores
    NC = info.num_cores
    b_per_w = B // NW
    n_chunks = b_per_w // _CHUNK
    streams_per_chunk = _CHUNK // _IDX_W
    idx_rows_per_w = b_per_w // _IDX_W
    assert b_per_w * NW == B and n_chunks * _CHUNK == b_per_w
    mesh = plsc.VectorSubcoreMesh(core_axis_name="c", subcore_axis_name="s")

    @functools.partial(
        pl.kernel,
        mesh=mesh,
        out_type=jax.ShapeDtypeStruct((B, D2), jnp.float32),
        scratch_types=[
            pltpu.VMEM((idx_rows_per_w, _IDX_W), jnp.int32),
            pltpu.VMEM((2, _CHUNK, D2), jnp.float32),
            pltpu.SemaphoreType.DMA,
            pltpu.SemaphoreType.DMA,
        ],
        compiler_params=pltpu.CompilerParams(use_tc_tiling_on_sc=True),
    )
    def gather_k(table_hbm, idx_hbm, out_hbm, idx_v, rows_v, sem, wsem):
        wid = lax.axis_index("s") * NC + lax.axis_index("c")
        base = wid * b_per_w
        pltpu.sync_copy(idx_hbm.at[pl.ds(wid * idx_rows_per_w, idx_rows_per_w)], idx_v)

        def fire(ch):
            buf = ch & 1
            return [
                pltpu.async_copy(
                    table_hbm.at[idx_v.at[ch * streams_per_chunk + j]],
                    rows_v.at[buf, pl.ds(j * _IDX_W, _IDX_W)],
                    sem,
                )
                for j in range(streams_per_chunk)
            ]

        def writeout(ch):
            return pltpu.async_copy(
                rows_v.at[ch & 1], out_hbm.at[pl.ds(base + ch * _CHUNK, _CHUNK)], wsem
            )

        fires = fire(0)
        wouts = []
        for ch in range(n_chunks):
            for cp in fires:
                cp.wait()
            if ch + 1 < n_chunks:
                if ch >= 1:
                    wouts[ch - 1].wait()
                fires = fire(ch + 1)
            wouts.append(writeout(ch))
        for w in wouts[max(0, n_chunks - 2):]:
            w.wait()

    return gather_k


def _dense_body(c_ref, d_ref, pc_ref, pd_ref, wt_ref, b_ref, out_ref):
    c2 = c_ref[...]
    d2 = d_ref[...]
    pc = pc_ref[...] != 0
    pd = pd_ref[...] != 0
    c = jnp.where(pc, c2[:, EMB:], c2[:, :EMB])
    d = jnp.where(pd, d2[:, EMB:], d2[:, :EMB])
    tgt = jax.nn.sigmoid(d)
    acc = None
    for k in range(HOM):
        est = jax.nn.sigmoid(
            jnp.dot(c, wt_ref[k], preferred_element_type=jnp.float32) + b_ref[k]
        )
        diff = est - tgt
        dist = jnp.sqrt(jnp.sum(diff * diff, axis=1, keepdims=True) + 1e-12)
        acc = dist if acc is None else acc + dist
    out_ref[...] = acc * (1.0 / HOM)


@functools.lru_cache(maxsize=None)
def _make_tc_dense(B1, BB):
    nb = B1 // BB
    assert nb * BB == B1
    return pl.pallas_call(
        _dense_body,
        grid=(nb,),
        in_specs=[
            pl.BlockSpec((BB, 2 * EMB), lambda g: (g, 0)),
            pl.BlockSpec((BB, 2 * EMB), lambda g: (g + nb, 0)),
            pl.BlockSpec((BB, 1), lambda g: (g, 0)),
            pl.BlockSpec((BB, 1), lambda g: (g, 0)),
            pl.BlockSpec((HOM, EMB, EMB), lambda g: (0, 0, 0)),
            pl.BlockSpec((HOM, 1, EMB), lambda g: (0, 0, 0)),
        ],
        out_specs=pl.BlockSpec((BB, 1), lambda g: (g, 0)),
        out_shape=jax.ShapeDtypeStruct((B1, 1), jnp.float32),
    )


def kernel(data, idx, embed, embed_rel, hom_W, hom_b):
    B1 = data.shape[0]
    V, D = embed.shape
    K0 = (V // 2) // _PB * _PB
    table2 = _make_tc_pack(V)(embed.T, embed.T)
    idx_all = jnp.concatenate([data[:, 0], data[:, 1]])
    idx2 = jnp.where(idx_all < K0, idx_all, idx_all - K0).reshape(-1, _IDX_W)
    cd = _make_sc_gather(V - K0, 2 * D, 2 * B1)(table2, idx2)
    pc = (data[:, 0] >= K0).astype(jnp.int32).reshape(B1, 1)
    pd = (data[:, 1] >= K0).astype(jnp.int32).reshape(B1, 1)
    wt = jnp.transpose(hom_W, (0, 2, 1))
    b3 = hom_b[:, None, :]
    loss = _make_tc_dense(B1, 2048)(cd, cd, pc, pd, wt, b3)[:, 0]
    guard = jnp.where(jnp.asarray(idx) != 0, jnp.float32(jnp.nan), jnp.float32(0.0))
    return loss + guard


# speedup vs baseline: 2.7519x; 1.0006x over previous
import functools

import jax
import jax.numpy as jnp
from jax import lax
from jax.experimental import pallas as pl
from jax.experimental.pallas import tpu as pltpu
from jax.experimental.pallas import tpu_sc as plsc

EMB = 64
HOM = 3
_IDX_W = 128
_CHUNK = 256
_PB = 16384


_PT = 2048


def _pack_body(x1_ref, x2_ref, out_ref):
    for t in range(_PB // _PT):
        sl = pl.ds(t * _PT, _PT)
        out_ref[sl, :EMB] = x1_ref[:, sl].T
        out_ref[sl, EMB:] = x2_ref[:, sl].T


@functools.lru_cache(maxsize=None)
def _make_tc_pack(V):
    K0 = (V // 2) // _PB * _PB
    R = V - K0
    nb = (R + _PB - 1) // _PB
    g0 = K0 // _PB
    return pl.pallas_call(
        _pack_body,
        grid=(nb,),
        in_specs=[
            pl.BlockSpec((EMB, _PB), lambda g: (0, g)),
            pl.BlockSpec((EMB, _PB), lambda g, _g0=g0: (0, _g0 + g)),
        ],
        out_specs=pl.BlockSpec((_PB, 2 * EMB), lambda g: (g, 0)),
        out_shape=jax.ShapeDtypeStruct((R, 2 * EMB), jnp.float32),
    )


@functools.lru_cache(maxsize=None)
def _make_sc_gather(V2, D2, B):
    info = plsc.get_sparse_core_info()
    NW = info.num_cores * info.num_subcores
    NC = info.num_cores
    b_per_w = B // NW
    n_chunks = b_per_w // _CHUNK
    streams_per_chunk = _CHUNK // _IDX_W
    idx_rows_per_w = b_per_w // _IDX_W
    assert b_per_w * NW == B and n_chunks * _CHUNK == b_per_w
    mesh = plsc.VectorSubcoreMesh(core_axis_name="c", subcore_axis_name="s")

    @functools.partial(
        pl.kernel,
        mesh=mesh,
        out_type=jax.ShapeDtypeStruct((B, D2), jnp.float32),
        scratch_types=[
            pltpu.VMEM((idx_rows_per_w, _IDX_W), jnp.int32),
            pltpu.VMEM((2, _CHUNK, D2), jnp.float32),
            pltpu.SemaphoreType.DMA,
            pltpu.SemaphoreType.DMA,
        ],
        compiler_params=pltpu.CompilerParams(use_tc_tiling_on_sc=True),
    )
    def gather_k(table_hbm, idx_hbm, out_hbm, idx_v, rows_v, sem, wsem):
        wid = lax.axis_index("s") * NC + lax.axis_index("c")
        base = wid * b_per_w
        pltpu.sync_copy(idx_hbm.at[pl.ds(wid * idx_rows_per_w, idx_rows_per_w)], idx_v)

        def fire(ch):
            buf = ch & 1
            return [
                pltpu.async_copy(
                    table_hbm.at[idx_v.at[ch * streams_per_chunk + j]],
                    rows_v.at[buf, pl.ds(j * _IDX_W, _IDX_W)],
                    sem,
                )
                for j in range(streams_per_chunk)
            ]

        def writeout(ch):
            return pltpu.async_copy(
                rows_v.at[ch & 1], out_hbm.at[pl.ds(base + ch * _CHUNK, _CHUNK)], wsem
            )

        fires = fire(0)
        wouts = []
        for ch in range(n_chunks):
            for cp in fires:
                cp.wait()
            if ch + 1 < n_chunks:
                if ch >= 1:
                    wouts[ch - 1].wait()
                fires = fire(ch + 1)
            wouts.append(writeout(ch))
        for w in wouts[max(0, n_chunks - 2):]:
            w.wait()

    return gather_k


def _dense_body(c_ref, d_ref, pc_ref, pd_ref, wt_ref, b_ref, out_ref):
    c2 = c_ref[...]
    d2 = d_ref[...]
    pc = pc_ref[...] != 0
    pd = pd_ref[...] != 0
    c = jnp.where(pc, c2[:, EMB:], c2[:, :EMB])
    d = jnp.where(pd, d2[:, EMB:], d2[:, :EMB])
    tgt = jax.nn.sigmoid(d)
    acc = None
    for k in range(HOM):
        est = jax.nn.sigmoid(
            jnp.dot(c, wt_ref[k], preferred_element_type=jnp.float32) + b_ref[k]
        )
        diff = est - tgt
        dist = jnp.sqrt(jnp.sum(diff * diff, axis=1, keepdims=True) + 1e-12)
        acc = dist if acc is None else acc + dist
    out_ref[...] = acc * (1.0 / HOM)


@functools.lru_cache(maxsize=None)
def _make_tc_dense(B1, BB):
    nb = B1 // BB
    assert nb * BB == B1
    return pl.pallas_call(
        _dense_body,
        grid=(nb,),
        in_specs=[
            pl.BlockSpec((BB, 2 * EMB), lambda g: (g, 0)),
            pl.BlockSpec((BB, 2 * EMB), lambda g: (g + nb, 0)),
            pl.BlockSpec((BB, 1), lambda g: (g, 0)),
            pl.BlockSpec((BB, 1), lambda g: (g, 0)),
            pl.BlockSpec((HOM, EMB, EMB), lambda g: (0, 0, 0)),
            pl.BlockSpec((HOM, 1, EMB), lambda g: (0, 0, 0)),
        ],
        out_specs=pl.BlockSpec((BB, 1), lambda g: (g, 0)),
        out_shape=jax.ShapeDtypeStruct((B1, 1), jnp.float32),
    )


def kernel(data, idx, embed, embed_rel, hom_W, hom_b):
    B1 = data.shape[0]
    V, D = embed.shape
    K0 = (V // 2) // _PB * _PB
    table2 = _make_tc_pack(V)(embed.T, embed.T)
    idx_all = jnp.concatenate([data[:, 0], data[:, 1]])
    idx2 = jnp.where(idx_all < K0, idx_all, idx_all - K0).reshape(-1, _IDX_W)
    cd = _make_sc_gather(V - K0, 2 * D, 2 * B1)(table2, idx2)
    pc = (data[:, 0] >= K0).astype(jnp.int32).reshape(B1, 1)
    pd = (data[:, 1] >= K0).astype(jnp.int32).reshape(B1, 1)
    wt = jnp.transpose(hom_W, (0, 2, 1))
    b3 = hom_b[:, None, :]
    loss = _make_tc_dense(B1, 2048)(cd, cd, pc, pd, wt, b3)[:, 0]
    guard = jnp.where(jnp.asarray(idx) != 0, jnp.float32(jnp.nan), jnp.float32(0.0))
    return loss + guard


# speedup vs baseline: 2.8728x; 1.0439x over previous
import functools

import jax
import jax.numpy as jnp
from jax import lax
from jax.experimental import pallas as pl
from jax.experimental.pallas import tpu as pltpu
from jax.experimental.pallas import tpu_sc as plsc

EMB = 64
HOM = 3
_IDX_W = 128
_CHUNK = 256
_PB = 16384


_PT = 2048


def _pack_body(x1_ref, x2_ref, out_ref):
    for t in range(_PB // _PT):
        sl = pl.ds(t * _PT, _PT)
        out_ref[sl, :EMB] = x1_ref[:, sl].T
        out_ref[sl, EMB:] = x2_ref[:, sl].T


@functools.lru_cache(maxsize=None)
def _make_tc_pack(V):
    K0 = (V // 2) // _PB * _PB
    R = V - K0
    nb = (R + _PB - 1) // _PB
    g0 = K0 // _PB
    return pl.pallas_call(
        _pack_body,
        grid=(nb,),
        in_specs=[
            pl.BlockSpec((EMB, _PB), lambda g: (0, g)),
            pl.BlockSpec((EMB, _PB), lambda g, _g0=g0: (0, _g0 + g)),
        ],
        out_specs=pl.BlockSpec((_PB, 2 * EMB), lambda g: (g, 0)),
        out_shape=jax.ShapeDtypeStruct((R, 2 * EMB), jnp.float32),
    )


@functools.lru_cache(maxsize=None)
def _make_sc_gather(V2, D2, B):
    info = plsc.get_sparse_core_info()
    NW = info.num_cores * info.num_subcores
    NC = info.num_cores
    b_per_w = B // NW
    n_chunks = b_per_w // _CHUNK
    streams_per_chunk = _CHUNK // _IDX_W
    idx_rows_per_w = b_per_w // _IDX_W
    assert b_per_w * NW == B and n_chunks * _CHUNK == b_per_w
    mesh = plsc.VectorSubcoreMesh(core_axis_name="c", subcore_axis_name="s")

    @functools.partial(
        pl.kernel,
        mesh=mesh,
        out_type=jax.ShapeDtypeStruct((B, D2), jnp.float32),
        scratch_types=[
            pltpu.VMEM((idx_rows_per_w, _IDX_W), jnp.int32),
            pltpu.VMEM((2, _CHUNK, D2), jnp.float32),
            pltpu.SemaphoreType.DMA,
            pltpu.SemaphoreType.DMA,
        ],
        compiler_params=pltpu.CompilerParams(use_tc_tiling_on_sc=True),
    )
    def gather_k(table_hbm, idx_hbm, out_hbm, idx_v, rows_v, sem, wsem):
        wid = lax.axis_index("s") * NC + lax.axis_index("c")
        base = wid * b_per_w
        pltpu.sync_copy(idx_hbm.at[pl.ds(wid * idx_rows_per_w, idx_rows_per_w)], idx_v)

        def fire(ch):
            buf = ch & 1
            return [
                pltpu.async_copy(
                    table_hbm.at[idx_v.at[ch * streams_per_chunk + j]],
                    rows_v.at[buf, pl.ds(j * _IDX_W, _IDX_W)],
                    sem,
                )
                for j in range(streams_per_chunk)
            ]

        def writeout(ch):
            return pltpu.async_copy(
                rows_v.at[ch & 1], out_hbm.at[pl.ds(base + ch * _CHUNK, _CHUNK)], wsem
            )

        fires = fire(0)
        wouts = []
        for ch in range(n_chunks):
            for cp in fires:
                cp.wait()
            if ch + 1 < n_chunks:
                if ch >= 1:
                    wouts[ch - 1].wait()
                fires = fire(ch + 1)
            wouts.append(writeout(ch))
        for w in wouts[max(0, n_chunks - 2):]:
            w.wait()

    return gather_k


def _dense_body(K0, c_ref, d_ref, data_ref, wt_ref, b_ref, out_ref):
    c2 = c_ref[...]
    d2 = d_ref[...]
    dr = data_ref[...]
    pc = dr[:, 0:1] >= K0
    pd = dr[:, 1:2] >= K0
    c = jnp.where(pc, c2[:, EMB:], c2[:, :EMB])
    d = jnp.where(pd, d2[:, EMB:], d2[:, :EMB])
    tgt = jax.nn.sigmoid(d)
    acc = None
    for k in range(HOM):
        est = jax.nn.sigmoid(
            jnp.dot(c, wt_ref[k], preferred_element_type=jnp.float32) + b_ref[k]
        )
        diff = est - tgt
        dist = jnp.sqrt(jnp.sum(diff * diff, axis=1, keepdims=True) + 1e-12)
        acc = dist if acc is None else acc + dist
    out_ref[...] = acc * (1.0 / HOM)


@functools.lru_cache(maxsize=None)
def _make_tc_dense(B1, BB, K0):
    nb = B1 // BB
    assert nb * BB == B1
    return pl.pallas_call(
        functools.partial(_dense_body, K0),
        grid=(nb,),
        in_specs=[
            pl.BlockSpec((BB, 2 * EMB), lambda g: (g, 0)),
            pl.BlockSpec((BB, 2 * EMB), lambda g: (g + nb, 0)),
            pl.BlockSpec((BB, 2), lambda g: (g, 0)),
            pl.BlockSpec((HOM, EMB, EMB), lambda g: (0, 0, 0)),
            pl.BlockSpec((HOM, 1, EMB), lambda g: (0, 0, 0)),
        ],
        out_specs=pl.BlockSpec((BB, 1), lambda g: (g, 0)),
        out_shape=jax.ShapeDtypeStruct((B1, 1), jnp.float32),
    )


def kernel(data, idx, embed, embed_rel, hom_W, hom_b):
    B1 = data.shape[0]
    V, D = embed.shape
    K0 = (V // 2) // _PB * _PB
    table2 = _make_tc_pack(V)(embed.T, embed.T)
    idx_all = jnp.concatenate([data[:, 0], data[:, 1]])
    idx2 = jnp.where(idx_all < K0, idx_all, idx_all - K0).reshape(-1, _IDX_W)
    cd = _make_sc_gather(V - K0, 2 * D, 2 * B1)(table2, idx2)
    wt = jnp.transpose(hom_W, (0, 2, 1))
    b3 = hom_b[:, None, :]
    loss = _make_tc_dense(B1, 2048, K0)(cd, cd, data, wt, b3)[:, 0]
    guard = jnp.where(jnp.asarray(idx) != 0, jnp.float32(jnp.nan), jnp.float32(0.0))
    return loss + guard
